# baseline (speedup 1.0000x reference)
import jax
import jax.numpy as jnp
from jax import lax
from jax.experimental import pallas as pl
from jax.experimental.pallas import tpu as pltpu
from jax.experimental.pallas import tpu_sc as plsc

VOCAB = 1000000
HID = 64
HID2 = 2 * HID
MAXPOS = 512
NTYPES = 2
B = 4096
L = 200

NC = 2
NS = 16
NW = NC * NS
RPW = B // NW

_INV_HID = 1.0 / HID
_EPS = 1e-12


def _rsqrt(x):
    xi = lax.bitcast_convert_type(x, jnp.int32)
    yi = jnp.int32(0x5F3759DF) - lax.shift_right_arithmetic(xi, 1)
    y = lax.bitcast_convert_type(yi, jnp.float32)
    half = jnp.float32(0.5) * x
    for _ in range(3):
        y = y * (jnp.float32(1.5) - half * y * y)
    return y


def _sc_body(wids, pids, tids, word, pos, typ, gamma, beta, out,
             widx, pidx, tidx, cidxb, wrows, ptrows,
             ptmp, ttmp, pttmp, gvec, bvec, pt_shared,
             sem_ids, sem_w, sem_pt, sem_out):
    c = lax.axis_index("c")
    s = lax.axis_index("s")
    wid = c * NS + s
    row0 = wid * RPW

    pltpu.sync_copy(pos.at[pl.ds(s * 32, 32)], ptmp)
    pltpu.sync_copy(typ, ttmp)
    pltpu.sync_copy(gamma, gvec)
    pltpu.sync_copy(beta, bvec)
    t0 = [ttmp[0, pl.ds(k * 16, 16)] for k in range(4)]
    t1 = [ttmp[1, pl.ds(k * 16, 16)] for k in range(4)]
    for r in range(32):
        for k in range(4):
            v = ptmp[r, pl.ds(k * 16, 16)]
            pttmp[2 * r, pl.ds(k * 16, 16)] = v + t0[k]
            pttmp[2 * r + 1, pl.ds(k * 16, 16)] = v + t1[k]
    pltpu.sync_copy(pttmp, pt_shared.at[pl.ds(s * 64, 64)])
    plsc.subcore_barrier()

    gv = [gvec[pl.ds(k * 16, 16)] for k in range(4)]
    bv = [bvec[pl.ds(k * 16, 16)] for k in range(4)]

    def ids_start(r, b):
        pltpu.async_copy(wids.at[r], widx.at[b], sem_ids[b])
        pltpu.async_copy(pids.at[r], pidx.at[b], sem_ids[b])
        pltpu.async_copy(tids.at[r], tidx.at[b], sem_ids[b])

    def ids_wait(b):
        pltpu.make_async_copy(wids.at[0], widx.at[b], sem_ids[b]).wait()
        pltpu.make_async_copy(pids.at[0], pidx.at[b], sem_ids[b]).wait()
        pltpu.make_async_copy(tids.at[0], tidx.at[b], sem_ids[b]).wait()

    _GOFF = [g * 16 for g in range(12)] + [L - 16]

    def cidx_compute(b):
        for off in _GOFF:
            p = pidx[b, pl.ds(off, 16)]
            t = tidx[b, pl.ds(off, 16)]
            cidxb[b, pl.ds(off, 16)] = p + p + t

    _SPLITS = ((0, 128), (128, L - 128))

    def gathers_start(b):
        for off, n in _SPLITS:
            pltpu.async_copy(word.at[widx.at[b, pl.ds(off, n)]],
                             wrows.at[b, pl.ds(off, n)], sem_w[b])
            pltpu.async_copy(pt_shared.at[cidxb.at[b, pl.ds(off, n)]],
                             ptrows.at[b, pl.ds(off, n)], sem_pt[b])

    def gathers_wait(b):
        for off, n in _SPLITS:
            pltpu.make_async_copy(word.at[widx.at[b, pl.ds(off, n)]],
                                  wrows.at[b, pl.ds(off, n)], sem_w[b]).wait()
            pltpu.make_async_copy(pt_shared.at[cidxb.at[b, pl.ds(off, n)]],
                                  ptrows.at[b, pl.ds(off, n)],
                                  sem_pt[b]).wait()

    def ln(b):
        @plsc.parallel_loop(0, L // 4)
        def _(j):
            for u in range(4):
                tok = j * 4 + u
                sv = [wrows[b, tok, pl.ds(k * 16, 16)]
                      + ptrows[b, tok, pl.ds(k * 16, 16)] for k in range(4)]
                tot = jnp.sum(sv[0] + sv[1] + sv[2] + sv[3])
                q = (sv[0] * sv[0] + sv[1] * sv[1]
                     + sv[2] * sv[2] + sv[3] * sv[3])
                ssq = jnp.sum(q)
                mu = tot * jnp.float32(_INV_HID)
                var = ssq * jnp.float32(_INV_HID) - mu * mu
                rstd = _rsqrt(var + jnp.float32(_EPS))
                for k in range(4):
                    wrows[b, tok, pl.ds(k * 16, 16)] = (
                        (sv[k] - mu) * rstd * gv[k] + bv[k])

    def out_start(r, b):
        pltpu.async_copy(wrows.at[b], out.at[r], sem_out[b])

    def out_wait(b):
        pltpu.make_async_copy(wrows.at[b], out.at[0], sem_out[b]).wait()

    pltpu.sync_copy(wids.at[row0], widx.at[0])
    pltpu.sync_copy(pids.at[row0], pidx.at[0])
    pltpu.sync_copy(tids.at[row0], tidx.at[0])
    cidx_compute(0)
    gathers_start(0)
    ids_start(row0 + 1, 1)

    gathers_wait(0)
    ids_start(row0 + 2, 0)
    ids_wait(1)
    cidx_compute(1)
    gathers_start(1)
    ln(0)
    out_start(row0, 0)

    gathers_wait(1)
    ids_start(row0 + 3, 1)
    ids_wait(0)
    cidx_compute(0)
    out_wait(0)
    gathers_start(0)
    ln(1)
    out_start(row0 + 1, 1)

    def steady(k, carry):
        rr = row0 + 2 + 2 * k
        for b in range(2):
            gathers_wait(b)
            ids_start(rr + b + 2, b)
            ids_wait(1 - b)
            cidx_compute(1 - b)
            out_wait(1 - b)
            gathers_start(1 - b)
            ln(b)
            out_start(rr + b, b)
        return carry

    lax.fori_loop(0, (RPW - 4) // 2, steady, 0, unroll=False)

    gathers_wait(0)
    ids_wait(1)
    cidx_compute(1)
    out_wait(1)
    gathers_start(1)
    ln(0)
    out_start(row0 + RPW - 2, 0)

    gathers_wait(1)
    ln(1)
    out_start(row0 + RPW - 1, 1)

    out_wait(0)
    out_wait(1)


@jax.jit
def _run(wids, pids, tids, word, pos, typ, gamma, beta):
    mesh = plsc.VectorSubcoreMesh(core_axis_name="c", subcore_axis_name="s")
    f = pl.kernel(
        _sc_body,
        out_type=jax.ShapeDtypeStruct((B, L, HID2), jnp.float32),
        mesh=mesh,
        compiler_params=pltpu.CompilerParams(
            needs_layout_passes=False, use_tc_tiling_on_sc=False),
        scratch_types=[
            pltpu.VMEM((2, L), jnp.int32),
            pltpu.VMEM((2, L), jnp.int32),
            pltpu.VMEM((2, L), jnp.int32),
            pltpu.VMEM((2, L), jnp.int32),
            pltpu.VMEM((2, L, HID2), jnp.float32),
            pltpu.VMEM((2, L, HID), jnp.float32),
            pltpu.VMEM((32, HID), jnp.float32),
            pltpu.VMEM((2, HID), jnp.float32),
            pltpu.VMEM((64, HID), jnp.float32),
            pltpu.VMEM((HID,), jnp.float32),
            pltpu.VMEM((HID,), jnp.float32),
            pltpu.VMEM_SHARED((MAXPOS * NTYPES, HID), jnp.float32),
            [pltpu.SemaphoreType.DMA, pltpu.SemaphoreType.DMA],
            [pltpu.SemaphoreType.DMA, pltpu.SemaphoreType.DMA],
            [pltpu.SemaphoreType.DMA, pltpu.SemaphoreType.DMA],
            [pltpu.SemaphoreType.DMA, pltpu.SemaphoreType.DMA],
        ],
    )
    return f(wids, pids, tids, word, pos, typ, gamma, beta)


def kernel(input_ids, position_ids, token_type_ids, word_emb, pos_emb,
           type_emb, gamma, beta):
    word_pad = jnp.pad(word_emb, ((0, 0), (0, HID)))
    out128 = _run(input_ids.astype(jnp.int32), position_ids.astype(jnp.int32),
                  token_type_ids.astype(jnp.int32), word_pad, pos_emb,
                  type_emb, gamma, beta)
    return out128[:, :, :HID]


# speedup vs baseline: 1.0369x; 1.0369x over previous
import jax
import jax.numpy as jnp
from jax import lax
from jax.experimental import pallas as pl
from jax.experimental.pallas import tpu as pltpu
from jax.experimental.pallas import tpu_sc as plsc

VOCAB = 1000000
HID = 64
HID2 = 2 * HID
MAXPOS = 512
NTYPES = 2
B = 4096
L = 200

NC = 2
NS = 16
NW = NC * NS
RPW = B // NW

_INV_HID = 1.0 / HID
_EPS = 1e-12


def _rsqrt(x):
    xi = lax.bitcast_convert_type(x, jnp.int32)
    yi = jnp.int32(0x5F3759DF) - lax.shift_right_arithmetic(xi, 1)
    y = lax.bitcast_convert_type(yi, jnp.float32)
    half = jnp.float32(0.5) * x
    for _ in range(3):
        y = y * (jnp.float32(1.5) - half * y * y)
    return y


def _sc_body(wids, pids, tids, word, pos, typ, gamma, beta, out,
             widx, pidx, tidx, cidxb, wrows, ptrows,
             ptmp, ttmp, pttmp, gvec, bvec, pt_shared,
             sem_ids, sem_w, sem_pt, sem_out):
    c = lax.axis_index("c")
    s = lax.axis_index("s")
    wid = c * NS + s
    row0 = wid * RPW

    pltpu.sync_copy(pos.at[pl.ds(s * 32, 32)], ptmp)
    pltpu.sync_copy(typ, ttmp)
    pltpu.sync_copy(gamma, gvec)
    pltpu.sync_copy(beta, bvec)
    t0 = [ttmp[0, pl.ds(k * 16, 16)] for k in range(4)]
    t1 = [ttmp[1, pl.ds(k * 16, 16)] for k in range(4)]
    for r in range(32):
        for k in range(4):
            v = ptmp[r, pl.ds(k * 16, 16)]
            pttmp[2 * r, pl.ds(k * 16, 16)] = v + t0[k]
            pttmp[2 * r + 1, pl.ds(k * 16, 16)] = v + t1[k]
    pltpu.sync_copy(pttmp, pt_shared.at[pl.ds(s * 64, 64)])
    plsc.subcore_barrier()

    gv = [gvec[pl.ds(k * 16, 16)] for k in range(4)]
    bv = [bvec[pl.ds(k * 16, 16)] for k in range(4)]

    def ids_start(r, b):
        pltpu.async_copy(wids.at[r], widx.at[b], sem_ids[b])
        pltpu.async_copy(pids.at[r], pidx.at[b], sem_ids[b])
        pltpu.async_copy(tids.at[r], tidx.at[b], sem_ids[b])

    def ids_wait(b):
        pltpu.make_async_copy(wids.at[0], widx.at[b], sem_ids[b]).wait()
        pltpu.make_async_copy(pids.at[0], pidx.at[b], sem_ids[b]).wait()
        pltpu.make_async_copy(tids.at[0], tidx.at[b], sem_ids[b]).wait()

    _GOFF = [g * 16 for g in range(12)] + [L - 16]

    def cidx_compute(b):
        for off in _GOFF:
            p = pidx[b, pl.ds(off, 16)]
            t = tidx[b, pl.ds(off, 16)]
            cidxb[b, pl.ds(off, 16)] = p + p + t

    _SPLITS = ((0, 128), (128, L - 128))

    def gathers_start(b):
        for off, n in _SPLITS:
            pltpu.async_copy(word.at[widx.at[b, pl.ds(off, n)]],
                             wrows.at[b, pl.ds(off, n)], sem_w[b])
            pltpu.async_copy(pt_shared.at[cidxb.at[b, pl.ds(off, n)]],
                             ptrows.at[b, pl.ds(off, n)], sem_pt[b])

    def gathers_wait(b):
        for off, n in _SPLITS:
            pltpu.make_async_copy(word.at[widx.at[b, pl.ds(off, n)]],
                                  wrows.at[b, pl.ds(off, n)], sem_w[b]).wait()
            pltpu.make_async_copy(pt_shared.at[cidxb.at[b, pl.ds(off, n)]],
                                  ptrows.at[b, pl.ds(off, n)],
                                  sem_pt[b]).wait()

    def ln(b):
        @plsc.parallel_loop(0, L // 4)
        def _(j):
            for u in range(4):
                tok = j * 4 + u
                sv = [wrows[b, tok, pl.ds(k * 16, 16)]
                      + ptrows[b, tok, pl.ds(k * 16, 16)] for k in range(4)]
                tot = jnp.sum(sv[0] + sv[1] + sv[2] + sv[3])
                q = (sv[0] * sv[0] + sv[1] * sv[1]
                     + sv[2] * sv[2] + sv[3] * sv[3])
                ssq = jnp.sum(q)
                mu = tot * jnp.float32(_INV_HID)
                var = ssq * jnp.float32(_INV_HID) - mu * mu
                rstd = _rsqrt(var + jnp.float32(_EPS))
                for k in range(4):
                    wrows[b, tok, pl.ds(k * 16, 16)] = (
                        (sv[k] - mu) * rstd * gv[k] + bv[k])

    def out_start(r, b):
        pltpu.async_copy(wrows.at[b], out.at[r], sem_out[b])

    def out_wait(b):
        pltpu.make_async_copy(wrows.at[b], out.at[0], sem_out[b]).wait()

    pltpu.sync_copy(wids.at[row0], widx.at[0])
    pltpu.sync_copy(pids.at[row0], pidx.at[0])
    pltpu.sync_copy(tids.at[row0], tidx.at[0])
    cidx_compute(0)
    gathers_start(0)
    ids_start(row0 + 1, 1)

    gathers_wait(0)
    ids_start(row0 + 2, 0)
    ids_wait(1)
    cidx_compute(1)
    gathers_start(1)
    ln(0)
    out_start(row0, 0)

    gathers_wait(1)
    ids_start(row0 + 3, 1)
    ids_wait(0)
    cidx_compute(0)
    out_wait(0)
    gathers_start(0)
    ln(1)
    out_start(row0 + 1, 1)

    def steady(k, carry):
        rr = row0 + 2 + 2 * k
        for b in range(2):
            gathers_wait(b)
            ids_start(rr + b + 2, b)
            ids_wait(1 - b)
            cidx_compute(1 - b)
            out_wait(1 - b)
            gathers_start(1 - b)
            ln(b)
            out_start(rr + b, b)
        return carry

    lax.fori_loop(0, (RPW - 4) // 2, steady, 0, unroll=False)

    gathers_wait(0)
    ids_wait(1)
    cidx_compute(1)
    out_wait(1)
    gathers_start(1)
    ln(0)
    out_start(row0 + RPW - 2, 0)

    gathers_wait(1)
    ln(1)
    out_start(row0 + RPW - 1, 1)

    out_wait(0)
    out_wait(1)


_TBLK = 2048


def _tr_body(in_ref, out_ref):
    x = in_ref[...]
    out_ref[:, :HID] = x.T
    out_ref[:, HID:] = jnp.zeros((_TBLK, HID), jnp.float32)


def _transpose_pad(word_t):
    grid = pl.cdiv(VOCAB, _TBLK)
    return pl.pallas_call(
        _tr_body,
        out_shape=jax.ShapeDtypeStruct((VOCAB, HID2), jnp.float32),
        grid=(grid,),
        in_specs=[pl.BlockSpec((HID, _TBLK), lambda j: (0, j))],
        out_specs=pl.BlockSpec((_TBLK, HID2), lambda j: (j, 0)),
    )(word_t)


@jax.jit
def _run(wids, pids, tids, word, pos, typ, gamma, beta):
    mesh = plsc.VectorSubcoreMesh(core_axis_name="c", subcore_axis_name="s")
    f = pl.kernel(
        _sc_body,
        out_type=jax.ShapeDtypeStruct((B, L, HID2), jnp.float32),
        mesh=mesh,
        compiler_params=pltpu.CompilerParams(
            needs_layout_passes=False, use_tc_tiling_on_sc=False),
        scratch_types=[
            pltpu.VMEM((2, L), jnp.int32),
            pltpu.VMEM((2, L), jnp.int32),
            pltpu.VMEM((2, L), jnp.int32),
            pltpu.VMEM((2, L), jnp.int32),
            pltpu.VMEM((2, L, HID2), jnp.float32),
            pltpu.VMEM((2, L, HID), jnp.float32),
            pltpu.VMEM((32, HID), jnp.float32),
            pltpu.VMEM((2, HID), jnp.float32),
            pltpu.VMEM((64, HID), jnp.float32),
            pltpu.VMEM((HID,), jnp.float32),
            pltpu.VMEM((HID,), jnp.float32),
            pltpu.VMEM_SHARED((MAXPOS * NTYPES, HID), jnp.float32),
            [pltpu.SemaphoreType.DMA, pltpu.SemaphoreType.DMA],
            [pltpu.SemaphoreType.DMA, pltpu.SemaphoreType.DMA],
            [pltpu.SemaphoreType.DMA, pltpu.SemaphoreType.DMA],
            [pltpu.SemaphoreType.DMA, pltpu.SemaphoreType.DMA],
        ],
    )
    return f(wids, pids, tids, word, pos, typ, gamma, beta)


def kernel(input_ids, position_ids, token_type_ids, word_emb, pos_emb,
           type_emb, gamma, beta):
    word_pad = _transpose_pad(word_emb.T)
    out128 = _run(input_ids.astype(jnp.int32), position_ids.astype(jnp.int32),
                  token_type_ids.astype(jnp.int32), word_pad, pos_emb,
                  type_emb, gamma, beta)
    return out128[:, :, :HID]


# speedup vs baseline: 1.2663x; 1.2212x over previous
import jax
import jax.numpy as jnp
from jax import lax
from jax.experimental import pallas as pl
from jax.experimental.pallas import tpu as pltpu
from jax.experimental.pallas import tpu_sc as plsc

VOCAB = 1000000
HID = 64
HID2 = 2 * HID
MAXPOS = 512
NTYPES = 2
B = 4096
L = 200

NC = 2
NS = 16
NW = NC * NS
RPW = B // NW

_INV_HID = 1.0 / HID
_EPS = 1e-12


def _rsqrt(x):
    xi = lax.bitcast_convert_type(x, jnp.int32)
    yi = jnp.int32(0x5F3759DF) - lax.shift_right_arithmetic(xi, 1)
    y = lax.bitcast_convert_type(yi, jnp.float32)
    half = jnp.float32(0.5) * x
    for _ in range(3):
        y = y * (jnp.float32(1.5) - half * y * y)
    return y


def _sc_body(wids, pids, tids, word, pos, typ, gamma, beta, out,
             widx, widx2, pidx, tidx, cidxb, wrows, ptrows, orows,
             ptmp, ttmp, pttmp, gvec, bvec, pt_shared,
             sem_ids, sem_w, sem_pt, sem_out):
    c = lax.axis_index("c")
    s = lax.axis_index("s")
    wid = c * NS + s
    row0 = wid * RPW

    pltpu.sync_copy(pos.at[pl.ds(s * 32, 32)], ptmp)
    pltpu.sync_copy(typ, ttmp)
    pltpu.sync_copy(gamma, gvec)
    pltpu.sync_copy(beta, bvec)
    t0 = [ttmp[0, pl.ds(k * 16, 16)] for k in range(4)]
    t1 = [ttmp[1, pl.ds(k * 16, 16)] for k in range(4)]
    for r in range(32):
        for k in range(4):
            v = ptmp[r, pl.ds(k * 16, 16)]
            pttmp[2 * r, pl.ds(k * 16, 16)] = v + t0[k]
            pttmp[2 * r + 1, pl.ds(k * 16, 16)] = v + t1[k]
    pltpu.sync_copy(pttmp, pt_shared.at[pl.ds(s * 64, 64)])
    plsc.subcore_barrier()

    gv = [gvec[pl.ds(k * 16, 16)] for k in range(4)]
    bv = [bvec[pl.ds(k * 16, 16)] for k in range(4)]

    def zero_pad(tok, carry):
        z = jnp.zeros((16,), jnp.float32)
        for bb in range(2):
            for k in range(4):
                orows[bb, tok, pl.ds(HID + k * 16, 16)] = z
        return carry

    lax.fori_loop(0, L, zero_pad, 0, unroll=False)

    def ids_start(r, b):
        pltpu.async_copy(wids.at[r], widx.at[b], sem_ids[b])
        pltpu.async_copy(pids.at[r], pidx.at[b], sem_ids[b])
        pltpu.async_copy(tids.at[r], tidx.at[b], sem_ids[b])

    def ids_wait(b):
        pltpu.make_async_copy(wids.at[0], widx.at[b], sem_ids[b]).wait()
        pltpu.make_async_copy(pids.at[0], pidx.at[b], sem_ids[b]).wait()
        pltpu.make_async_copy(tids.at[0], tidx.at[b], sem_ids[b]).wait()

    _GOFF = [g * 16 for g in range(12)] + [L - 16]

    def cidx_compute(b):
        for off in _GOFF:
            p = pidx[b, pl.ds(off, 16)]
            t = tidx[b, pl.ds(off, 16)]
            cidxb[b, pl.ds(off, 16)] = p + p + t
            v = widx[b, pl.ds(off, 16)]
            adj = jnp.where(v >= jnp.int32(1 << 19),
                            jnp.int32((1 << 20) - 1), jnp.int32(0))
            widx2[b, pl.ds(off, 16)] = v + v - adj

    _SPLITS = ((0, 128), (128, L - 128))

    def gathers_start(b):
        for off, n in _SPLITS:
            pltpu.async_copy(word.at[widx2.at[b, pl.ds(off, n)]],
                             wrows.at[b, pl.ds(off, n)], sem_w[b])
            pltpu.async_copy(pt_shared.at[cidxb.at[b, pl.ds(off, n)]],
                             ptrows.at[b, pl.ds(off, n)], sem_pt[b])

    def gathers_wait(b):
        for off, n in _SPLITS:
            pltpu.make_async_copy(word.at[widx2.at[b, pl.ds(off, n)]],
                                  wrows.at[b, pl.ds(off, n)], sem_w[b]).wait()
            pltpu.make_async_copy(pt_shared.at[cidxb.at[b, pl.ds(off, n)]],
                                  ptrows.at[b, pl.ds(off, n)],
                                  sem_pt[b]).wait()

    def ln(b):
        @plsc.parallel_loop(0, L // 4)
        def _(j):
            for u in range(4):
                tok = j * 4 + u
                sv = [wrows[b, tok, pl.ds(k * 16, 16)]
                      + ptrows[b, tok, pl.ds(k * 16, 16)] for k in range(4)]
                tot = jnp.sum(sv[0] + sv[1] + sv[2] + sv[3])
                q = (sv[0] * sv[0] + sv[1] * sv[1]
                     + sv[2] * sv[2] + sv[3] * sv[3])
                ssq = jnp.sum(q)
                mu = tot * jnp.float32(_INV_HID)
                var = ssq * jnp.float32(_INV_HID) - mu * mu
                rstd = _rsqrt(var + jnp.float32(_EPS))
                for k in range(4):
                    orows[b, tok, pl.ds(k * 16, 16)] = (
                        (sv[k] - mu) * rstd * gv[k] + bv[k])

    def out_start(r, b):
        pltpu.async_copy(orows.at[b], out.at[r], sem_out[b])

    def out_wait(b):
        pltpu.make_async_copy(orows.at[b], out.at[0], sem_out[b]).wait()

    pltpu.sync_copy(wids.at[row0], widx.at[0])
    pltpu.sync_copy(pids.at[row0], pidx.at[0])
    pltpu.sync_copy(tids.at[row0], tidx.at[0])
    cidx_compute(0)
    gathers_start(0)
    ids_start(row0 + 1, 1)

    gathers_wait(0)
    ids_start(row0 + 2, 0)
    ids_wait(1)
    cidx_compute(1)
    gathers_start(1)
    ln(0)
    out_start(row0, 0)

    gathers_wait(1)
    ids_start(row0 + 3, 1)
    ids_wait(0)
    cidx_compute(0)
    gathers_start(0)
    ln(1)
    out_start(row0 + 1, 1)

    def steady(k, carry):
        rr = row0 + 2 + 2 * k
        for b in range(2):
            gathers_wait(b)
            ids_start(rr + b + 2, b)
            ids_wait(1 - b)
            cidx_compute(1 - b)
            gathers_start(1 - b)
            out_wait(b)
            ln(b)
            out_start(rr + b, b)
        return carry

    lax.fori_loop(0, (RPW - 4) // 2, steady, 0, unroll=False)

    gathers_wait(0)
    ids_wait(1)
    cidx_compute(1)
    gathers_start(1)
    out_wait(0)
    ln(0)
    out_start(row0 + RPW - 2, 0)

    gathers_wait(1)
    out_wait(1)
    ln(1)
    out_start(row0 + RPW - 1, 1)

    out_wait(0)
    out_wait(1)


_VP = 1 << 20
_TBLK = 2048
_NTB = (_VP // 2) // _TBLK


def _tr_body(lo_ref, hi_ref, out_ref):
    out_ref[:, :HID] = lo_ref[...].T
    out_ref[:, HID:] = hi_ref[...].T


def _word_prep(word_t):
    return pl.pallas_call(
        _tr_body,
        out_shape=jax.ShapeDtypeStruct((_VP // 2, HID2), jnp.float32),
        grid=(_NTB,),
        in_specs=[pl.BlockSpec((HID, _TBLK), lambda j: (0, j)),
                  pl.BlockSpec(
                      (HID, _TBLK),
                      lambda j: (0, jnp.minimum(
                          j + _NTB, (VOCAB + _TBLK - 1) // _TBLK - 1)))],
        out_specs=pl.BlockSpec((_TBLK, HID2), lambda j: (j, 0)),
    )(word_t, word_t)


@jax.jit
def _run(wids, pids, tids, word, pos, typ, gamma, beta):
    mesh = plsc.VectorSubcoreMesh(core_axis_name="c", subcore_axis_name="s")
    f = pl.kernel(
        _sc_body,
        out_type=jax.ShapeDtypeStruct((B, L, HID2), jnp.float32),
        mesh=mesh,
        compiler_params=pltpu.CompilerParams(
            needs_layout_passes=False, use_tc_tiling_on_sc=False),
        scratch_types=[
            pltpu.VMEM((2, L), jnp.int32),
            pltpu.VMEM((2, L), jnp.int32),
            pltpu.VMEM((2, L), jnp.int32),
            pltpu.VMEM((2, L), jnp.int32),
            pltpu.VMEM((2, L), jnp.int32),
            pltpu.VMEM((2, L, HID), jnp.float32),
            pltpu.VMEM((2, L, HID), jnp.float32),
            pltpu.VMEM((2, L, HID2), jnp.float32),
            pltpu.VMEM((32, HID), jnp.float32),
            pltpu.VMEM((2, HID), jnp.float32),
            pltpu.VMEM((64, HID), jnp.float32),
            pltpu.VMEM((HID,), jnp.float32),
            pltpu.VMEM((HID,), jnp.float32),
            pltpu.VMEM_SHARED((MAXPOS * NTYPES, HID), jnp.float32),
            [pltpu.SemaphoreType.DMA, pltpu.SemaphoreType.DMA],
            [pltpu.SemaphoreType.DMA, pltpu.SemaphoreType.DMA],
            [pltpu.SemaphoreType.DMA, pltpu.SemaphoreType.DMA],
            [pltpu.SemaphoreType.DMA, pltpu.SemaphoreType.DMA],
        ],
    )
    return f(wids, pids, tids, word, pos, typ, gamma, beta)


def kernel(input_ids, position_ids, token_type_ids, word_emb, pos_emb,
           type_emb, gamma, beta):
    word_lin = _word_prep(word_emb.T).reshape(_VP, HID)
    out128 = _run(input_ids.astype(jnp.int32), position_ids.astype(jnp.int32),
                  token_type_ids.astype(jnp.int32), word_lin, pos_emb,
                  type_emb, gamma, beta)
    return out128[:, :, :HID]


# speedup vs baseline: 1.3758x; 1.0864x over previous
import jax
import jax.numpy as jnp
from jax import lax
from jax.experimental import pallas as pl
from jax.experimental.pallas import tpu as pltpu
from jax.experimental.pallas import tpu_sc as plsc

VOCAB = 1000000
HID = 64
HID2 = 2 * HID
MAXPOS = 512
NTYPES = 2
B = 4096
L = 200

NC = 2
NS = 16
NW = NC * NS
RPW = B // NW

_INV_HID = 1.0 / HID
_EPS = 1e-12


def _rsqrt(x):
    xi = lax.bitcast_convert_type(x, jnp.int32)
    yi = jnp.int32(0x5F3759DF) - lax.shift_right_arithmetic(xi, 1)
    y = lax.bitcast_convert_type(yi, jnp.float32)
    half = jnp.float32(0.5) * x
    for _ in range(3):
        y = y * (jnp.float32(1.5) - half * y * y)
    return y


def _sc_body(wids, pids, tids, word, pos, typ, gamma, beta, out,
             widx, widx2, pidx, tidx, cidxb, wrows, ptrows, orows,
             ptmp, ttmp, pttmp, gvec, bvec, pt_shared,
             sem_ids, sem_w, sem_pt, sem_out):
    c = lax.axis_index("c")
    s = lax.axis_index("s")
    wid = c * NS + s
    row0 = wid * RPW

    pltpu.sync_copy(pos.at[pl.ds(s * 32, 32)], ptmp)
    pltpu.sync_copy(typ, ttmp)
    pltpu.sync_copy(gamma, gvec)
    pltpu.sync_copy(beta, bvec)
    t0 = [ttmp[0, pl.ds(k * 16, 16)] for k in range(4)]
    t1 = [ttmp[1, pl.ds(k * 16, 16)] for k in range(4)]
    for r in range(32):
        for k in range(4):
            v = ptmp[r, pl.ds(k * 16, 16)]
            pttmp[2 * r, pl.ds(k * 16, 16)] = v + t0[k]
            pttmp[2 * r + 1, pl.ds(k * 16, 16)] = v + t1[k]
    pltpu.sync_copy(pttmp, pt_shared.at[pl.ds(s * 64, 64)])
    plsc.subcore_barrier()

    gv = [gvec[pl.ds(k * 16, 16)] for k in range(4)]
    bv = [bvec[pl.ds(k * 16, 16)] for k in range(4)]

    def zero_pad(tok, carry):
        z = jnp.zeros((16,), jnp.float32)
        for bb in range(2):
            for k in range(4):
                orows[bb, tok, pl.ds(HID + k * 16, 16)] = z
        return carry

    lax.fori_loop(0, L, zero_pad, 0, unroll=False)

    def ids_start(r, b):
        pltpu.async_copy(wids.at[r], widx.at[b], sem_ids[b])
        pltpu.async_copy(pids.at[r], pidx.at[b], sem_ids[b])
        pltpu.async_copy(tids.at[r], tidx.at[b], sem_ids[b])

    def ids_wait(b):
        pltpu.make_async_copy(wids.at[0], widx.at[b], sem_ids[b]).wait()
        pltpu.make_async_copy(pids.at[0], pidx.at[b], sem_ids[b]).wait()
        pltpu.make_async_copy(tids.at[0], tidx.at[b], sem_ids[b]).wait()

    _GOFF = [g * 16 for g in range(12)] + [L - 16]

    def cidx_compute(b):
        for off in _GOFF:
            p = pidx[b, pl.ds(off, 16)]
            t = tidx[b, pl.ds(off, 16)]
            cidxb[b, pl.ds(off, 16)] = p + p + t
            v = widx[b, pl.ds(off, 16)]
            adj = jnp.where(v >= jnp.int32(1 << 19),
                            jnp.int32((1 << 20) - 1), jnp.int32(0))
            widx2[b, pl.ds(off, 16)] = v + v - adj

    _SPLITS = ((0, 128), (128, L - 128))

    def gathers_start(b):
        for off, n in _SPLITS:
            pltpu.async_copy(word.at[widx2.at[b, pl.ds(off, n)]],
                             wrows.at[b, pl.ds(off, n)], sem_w[b])
            pltpu.async_copy(pt_shared.at[cidxb.at[b, pl.ds(off, n)]],
                             ptrows.at[b, pl.ds(off, n)], sem_pt[b])

    def gathers_wait(b):
        for off, n in _SPLITS:
            pltpu.make_async_copy(word.at[widx2.at[b, pl.ds(off, n)]],
                                  wrows.at[b, pl.ds(off, n)], sem_w[b]).wait()
            pltpu.make_async_copy(pt_shared.at[cidxb.at[b, pl.ds(off, n)]],
                                  ptrows.at[b, pl.ds(off, n)],
                                  sem_pt[b]).wait()

    def ln(b):
        @plsc.parallel_loop(0, L // 4)
        def _(j):
            for u in range(4):
                tok = j * 4 + u
                sv = [wrows[b, tok, pl.ds(k * 16, 16)]
                      + ptrows[b, tok, pl.ds(k * 16, 16)] for k in range(4)]
                tot = jnp.sum(sv[0] + sv[1] + sv[2] + sv[3])
                q = (sv[0] * sv[0] + sv[1] * sv[1]
                     + sv[2] * sv[2] + sv[3] * sv[3])
                ssq = jnp.sum(q)
                mu = tot * jnp.float32(_INV_HID)
                var = ssq * jnp.float32(_INV_HID) - mu * mu
                rstd = _rsqrt(var + jnp.float32(_EPS))
                for k in range(4):
                    orows[b, tok, pl.ds(k * 16, 16)] = (
                        (sv[k] - mu) * rstd * gv[k] + bv[k])

    def out_start(r, b):
        pltpu.async_copy(orows.at[b], out.at[r], sem_out[b])

    def out_wait(b):
        pltpu.make_async_copy(orows.at[b], out.at[0], sem_out[b]).wait()

    pltpu.sync_copy(wids.at[row0], widx.at[0])
    pltpu.sync_copy(pids.at[row0], pidx.at[0])
    pltpu.sync_copy(tids.at[row0], tidx.at[0])
    cidx_compute(0)
    gathers_start(0)
    ids_start(row0 + 1, 1)

    gathers_wait(0)
    ids_start(row0 + 2, 0)
    ids_wait(1)
    cidx_compute(1)
    gathers_start(1)
    ln(0)
    out_start(row0, 0)

    gathers_wait(1)
    ids_start(row0 + 3, 1)
    ids_wait(0)
    cidx_compute(0)
    gathers_start(0)
    ln(1)
    out_start(row0 + 1, 1)

    def steady(k, carry):
        rr = row0 + 2 + 2 * k
        for b in range(2):
            gathers_wait(b)
            ids_start(rr + b + 2, b)
            ids_wait(1 - b)
            cidx_compute(1 - b)
            gathers_start(1 - b)
            out_wait(b)
            ln(b)
            out_start(rr + b, b)
        return carry

    lax.fori_loop(0, (RPW - 4) // 2, steady, 0, unroll=False)

    gathers_wait(0)
    ids_wait(1)
    cidx_compute(1)
    gathers_start(1)
    out_wait(0)
    ln(0)
    out_start(row0 + RPW - 2, 0)

    gathers_wait(1)
    out_wait(1)
    ln(1)
    out_start(row0 + RPW - 1, 1)

    out_wait(0)
    out_wait(1)


_VP = 1 << 20
_TBLK = 4096
_NTB = (_VP // 2) // _TBLK


def _tr_body(lo_ref, hi_ref, out_ref):
    out_ref[...] = jnp.concatenate(
        [lo_ref[...].T, hi_ref[...].T], axis=1)


def _word_prep(word_t):
    return pl.pallas_call(
        _tr_body,
        out_shape=jax.ShapeDtypeStruct((_VP // 2, HID2), jnp.float32),
        grid=(_NTB,),
        in_specs=[pl.BlockSpec((HID, _TBLK), lambda j: (0, j)),
                  pl.BlockSpec(
                      (HID, _TBLK),
                      lambda j: (0, jnp.minimum(
                          j + _NTB, (VOCAB + _TBLK - 1) // _TBLK - 1)))],
        out_specs=pl.BlockSpec((_TBLK, HID2), lambda j: (j, 0)),
    )(word_t, word_t)


@jax.jit
def _run(wids, pids, tids, word, pos, typ, gamma, beta):
    mesh = plsc.VectorSubcoreMesh(core_axis_name="c", subcore_axis_name="s")
    f = pl.kernel(
        _sc_body,
        out_type=jax.ShapeDtypeStruct((B, L, HID2), jnp.float32),
        mesh=mesh,
        compiler_params=pltpu.CompilerParams(
            needs_layout_passes=False, use_tc_tiling_on_sc=False),
        scratch_types=[
            pltpu.VMEM((2, L), jnp.int32),
            pltpu.VMEM((2, L), jnp.int32),
            pltpu.VMEM((2, L), jnp.int32),
            pltpu.VMEM((2, L), jnp.int32),
            pltpu.VMEM((2, L), jnp.int32),
            pltpu.VMEM((2, L, HID), jnp.float32),
            pltpu.VMEM((2, L, HID), jnp.float32),
            pltpu.VMEM((2, L, HID2), jnp.float32),
            pltpu.VMEM((32, HID), jnp.float32),
            pltpu.VMEM((2, HID), jnp.float32),
            pltpu.VMEM((64, HID), jnp.float32),
            pltpu.VMEM((HID,), jnp.float32),
            pltpu.VMEM((HID,), jnp.float32),
            pltpu.VMEM_SHARED((MAXPOS * NTYPES, HID), jnp.float32),
            [pltpu.SemaphoreType.DMA, pltpu.SemaphoreType.DMA],
            [pltpu.SemaphoreType.DMA, pltpu.SemaphoreType.DMA],
            [pltpu.SemaphoreType.DMA, pltpu.SemaphoreType.DMA],
            [pltpu.SemaphoreType.DMA, pltpu.SemaphoreType.DMA],
        ],
    )
    return f(wids, pids, tids, word, pos, typ, gamma, beta)


def kernel(input_ids, position_ids, token_type_ids, word_emb, pos_emb,
           type_emb, gamma, beta):
    word_lin = _word_prep(word_emb.T).reshape(_VP, HID)
    out128 = _run(input_ids.astype(jnp.int32), position_ids.astype(jnp.int32),
                  token_type_ids.astype(jnp.int32), word_lin, pos_emb,
                  type_emb, gamma, beta)
    return out128[:, :, :HID]
